# Initial kernel scaffold; baseline (speedup 1.0000x reference)
import jax
import jax.numpy as jnp
from jax.experimental import pallas as pl


def kernel(feat0, feat1, warm_idx, inter_rows, inter_cols, W0, b0, g0, be0, W1, b1, g1, be1, fuse_w, Wf, bf):
    raise NotImplementedError("write your pallas kernel here")


# speedup vs baseline: 18.1848x; 18.1848x over previous
import functools
import jax
import jax.numpy as jnp
from jax import lax
from jax.experimental import pallas as pl
from jax.experimental.pallas import tpu as pltpu
from jax.experimental.pallas import tpu_sc as plsc

N_ITEMS = 50000
N_WARM = 40000
N_USERS = 50000
NNZ = 800000
D0, D1 = 512, 384
HID = 512
EMB = 64

BLK = 2000

NW = 32
U_PER_W = 1568
U_PER_SC = 16 * U_PER_W
U_PAD = NW * U_PER_W
TRASH = U_PER_SC
CHUNK = 128
NSLOT = 3
ALIGN = 2 * NSLOT * CHUNK
NNZ_PAD = ((NNZ + ALIGN - 1) // ALIGN) * ALIGN
W_PER_SUB = 1280
N_WARM_PAD = NW * W_PER_SUB


def _count_body(rows_ref, cnt_ref):
    cnt_ref[...] = jnp.concatenate([jnp.stack([
        jnp.sum((rows_ref[...] < t * U_PER_W).astype(jnp.float32))
        for t in range(NW + 1)])[None], jnp.zeros((1, 479), jnp.float32)],
        axis=1)


def _mlp_body(x0_ref, x1_ref, w0_ref, w1_ref, z0_ref, z1_ref,
              st_ref):
    x0 = x0_ref[...]
    n0 = jnp.sqrt(jnp.sum(x0 * x0, axis=1, keepdims=True))
    r0 = 1.0 / jnp.maximum(n0, 1e-12)
    zb0 = (jnp.dot(x0.astype(jnp.bfloat16), w0_ref[...],
                   preferred_element_type=jnp.float32)
           * r0).astype(jnp.bfloat16)
    z0_ref[...] = zb0
    x1 = x1_ref[...]
    n1 = jnp.sqrt(jnp.sum(x1 * x1, axis=1, keepdims=True))
    r1 = 1.0 / jnp.maximum(n1, 1e-12)
    zb1 = (jnp.dot(x1.astype(jnp.bfloat16), w1_ref[...],
                   preferred_element_type=jnp.float32)
           * r1).astype(jnp.bfloat16)
    z1_ref[...] = zb1
    z0 = zb0.astype(jnp.float32)
    z1 = zb1.astype(jnp.float32)
    st = jnp.concatenate(
        [jnp.sum(z0, axis=0)[None], jnp.sum(z0 * z0, axis=0)[None],
         jnp.sum(z1, axis=0)[None], jnp.sum(z1 * z1, axis=0)[None],
         jnp.zeros((4, HID), jnp.float32)], axis=0)

    @pl.when(pl.program_id(0) == 0)
    def _():
        st_ref[...] = st

    @pl.when(pl.program_id(0) > 0)
    def _():
        st_ref[...] += st


def _fuse_body(z0_ref, z1_ref, par_ref, wf_ref, bfp_ref, fn_ref):
    z0 = z0_ref[...].astype(jnp.float32)
    z1 = z1_ref[...].astype(jnp.float32)
    h0 = jnp.maximum(z0 * par_ref[0:1, :] + par_ref[1:2, :], 0.0)
    h1 = jnp.maximum(z1 * par_ref[2:3, :] + par_ref[3:4, :], 0.0)
    f = jnp.dot((h0 + h1).astype(jnp.bfloat16), wf_ref[...],
                preferred_element_type=jnp.float32) + bfp_ref[0:1, :]
    n = jnp.sqrt(jnp.sum(f * f, axis=1, keepdims=True))
    fn_ref[...] = f * (1.0 / jnp.maximum(n, 1e-12))


def _norm_body(x_ref, o_ref):
    x = x_ref[...]
    n = jnp.sqrt(jnp.sum(x * x, axis=1, keepdims=True))
    o_ref[...] = x * (1.0 / jnp.maximum(n, 1e-12))


@functools.lru_cache(maxsize=None)
def _sc_kernels():
    mesh = plsc.VectorSubcoreMesh(core_axis_name="c", subcore_axis_name="s")
    warm_gather = functools.partial(
        pl.kernel, mesh=mesh,
        out_type=jax.ShapeDtypeStruct((N_WARM_PAD, EMB), jnp.float32),
        scratch_types=[
            pltpu.VMEM((8, 128), jnp.int32),
            pltpu.VMEM((1024, EMB), jnp.float32),
            pltpu.SemaphoreType.DMA,
        ],
        compiler_params=pltpu.CompilerParams(use_tc_tiling_on_sc=False, needs_layout_passes=False),
    )(_warm_gather_body)
    edge_agg = functools.partial(
        pl.kernel, mesh=mesh,
        out_type=jax.ShapeDtypeStruct((U_PAD, EMB), jnp.float32),
        scratch_types=[
            pltpu.VMEM((4, 16), jnp.int32),
            [pltpu.VMEM((128,), jnp.int32)] * NSLOT,
            [pltpu.VMEM((128,), jnp.int32)] * NSLOT,
            [pltpu.VMEM((1, 128), jnp.int32)] * NSLOT,
            [pltpu.VMEM((CHUNK, EMB), jnp.float32)] * NSLOT,
            pltpu.VMEM_SHARED((U_PER_SC + 8, EMB), jnp.float32),
            [pltpu.SemaphoreType.DMA] * NSLOT,
            [pltpu.SemaphoreType.DMA] * NSLOT,
            [pltpu.SemaphoreType.DMA] * NSLOT,
        ],
        compiler_params=pltpu.CompilerParams(use_tc_tiling_on_sc=False, needs_layout_passes=False),
    )(_edge_agg_body)
    return warm_gather, edge_agg


def _warm_gather_body(fn_hbm, idx_hbm, warm_hbm, idx_v, gbuf, sem):
    w = lax.axis_index("c") * 16 + lax.axis_index("s")
    n_units = N_WARM_PAD // 1024

    def do_unit(u):
        pltpu.sync_copy(idx_hbm.at[pl.ds(u * 8, 8)], idx_v)
        descs = []
        for j in range(8):
            descs.append(pltpu.async_copy(
                fn_hbm.at[idx_v.at[j]], gbuf.at[pl.ds(j * 128, 128)], sem))
        for j in range(8):
            descs[j].wait()
        pltpu.sync_copy(gbuf, warm_hbm.at[pl.ds(u * 1024, 1024)])

    do_unit(w)

    @pl.when(w < n_units - NW)
    def _():
        do_unit(w + NW)


def _edge_agg_body(rows_hbm, cols_hbm, bounds_hbm, zer_hbm, warm_hbm, out_hbm,
                   bv, rv, cv, iv, gbuf, acc, sem_rc, sem_g, sem_s):
    cid = lax.axis_index("c")
    sid = lax.axis_index("s")
    w = cid * 16 + sid
    u_lo = w * U_PER_W
    u_hi = u_lo + U_PER_W
    acc_base = cid * U_PER_SC

    pltpu.sync_copy(zer_hbm, acc.at[pl.ds(sid * U_PER_W, U_PER_W)])

    @pl.when(sid == 0)
    def _():
        pltpu.sync_copy(zer_hbm.at[pl.ds(0, 8)], acc.at[pl.ds(U_PER_SC, 8)])

    pltpu.sync_copy(bounds_hbm, bv)
    lane = lax.iota(jnp.int32, 16)
    sv = jnp.where(cid == 0, bv[0, :], bv[1, :])
    nv = jnp.where(cid == 0, bv[2, :], bv[3, :])
    srow = jnp.max(jnp.where(lane == sid, sv, 0))
    nch = jnp.max(jnp.where(lane == sid, nv, 0))

    plsc.subcore_barrier()

    def fire_rc(j, p):
        pltpu.async_copy(rows_hbm.at[pl.ds((srow + j) * 128, 128)], rv[p],
                         sem_rc[p])
        pltpu.async_copy(cols_hbm.at[pl.ds((srow + j) * 128, 128)], cv[p],
                         sem_rc[p])

    def wait_rc(j, p):
        pltpu.make_async_copy(rows_hbm.at[pl.ds((srow + j) * 128, 128)],
                              rv[p], sem_rc[p]).wait()
        pltpu.make_async_copy(cols_hbm.at[pl.ds((srow + j) * 128, 128)],
                              cv[p], sem_rc[p]).wait()

    def drain_scatter(p):
        pltpu.make_async_copy(gbuf[p], acc.at[iv[p].at[0]], sem_s[p]).wait()

    for p in range(NSLOT):
        @pl.when(nch > p)
        def _(p=p):
            fire_rc(p, p)

    def phase1(j, p):
        @pl.when(j >= NSLOT)
        def _():
            drain_scatter(p)
        wait_rc(j, p)
        for k in range(8):
            r = rv[p][pl.ds(k * 16, 16)]
            owned = (r >= u_lo) & (r < u_hi)
            iv[p][0, pl.ds(k * 16, 16)] = jnp.where(owned, r - acc_base, TRASH)
        return pltpu.async_copy(warm_hbm.at[cv[p]], gbuf[p], sem_g[p])

    def phase2(gd, j, p):
        gd.wait()
        pltpu.async_copy(gbuf[p], acc.at[iv[p].at[0]], sem_s[p], add=True)

        @pl.when(j + NSLOT < nch)
        def _():
            fire_rc(j + NSLOT, p)

    def loop_body(j6, carry):
        base = 2 * NSLOT * j6
        pend = []
        for idx in range(2 * NSLOT):
            p = idx % NSLOT
            if len(pend) == NSLOT or (len(pend) == NSLOT - 1 and idx >= NSLOT):
                gd, jj, pp = pend.pop(0)
                phase2(gd, jj, pp)
            pend.append((phase1(base + idx, p), base + idx, p))
        for gd, jj, pp in pend:
            phase2(gd, jj, pp)
        return carry

    lax.fori_loop(0, nch // (2 * NSLOT), loop_body, 0)

    @pl.when(nch >= NSLOT)
    def _():
        for p in range(NSLOT):
            drain_scatter(p)

    plsc.subcore_barrier()
    pltpu.sync_copy(acc.at[pl.ds(sid * U_PER_W, U_PER_W)],
                    out_hbm.at[pl.ds(w * U_PER_W, U_PER_W)])


def kernel(feat0, feat1, warm_idx, inter_rows, inter_cols,
           W0, b0, g0, be0, W1, b1, g1, be1, fuse_w, Wf, bf):
    grid = N_ITEMS // BLK

    z0, z1, st = pl.pallas_call(
        _mlp_body,
        grid=(grid,),
        in_specs=[
            pl.BlockSpec((BLK, D0), lambda i: (i, 0)),
            pl.BlockSpec((BLK, D1), lambda i: (i, 0)),
            pl.BlockSpec((D0, HID), lambda i: (0, 0)),
            pl.BlockSpec((D1, HID), lambda i: (0, 0)),
        ],
        out_specs=[
            pl.BlockSpec((BLK, HID), lambda i: (i, 0)),
            pl.BlockSpec((BLK, HID), lambda i: (i, 0)),
            pl.BlockSpec((8, HID), lambda i: (0, 0)),
        ],
        out_shape=[
            jax.ShapeDtypeStruct((N_ITEMS, HID), jnp.bfloat16),
            jax.ShapeDtypeStruct((N_ITEMS, HID), jnp.bfloat16),
            jax.ShapeDtypeStruct((8, HID), jnp.float32),
        ],
    )(feat0, feat1, W0.astype(jnp.bfloat16), W1.astype(jnp.bfloat16))

    cnt = pl.pallas_call(
        _count_body,
        grid=(1,),
        in_specs=[pl.BlockSpec((NNZ,), lambda i: (0,))],
        out_specs=pl.BlockSpec((1, 512), lambda i: (0, 0)),
        out_shape=jax.ShapeDtypeStruct((1, 512), jnp.float32),
    )(inter_rows)

    inv_n = 1.0 / N_ITEMS
    m0 = st[0] * inv_n
    v0 = st[1] * inv_n - m0 * m0
    m1 = st[2] * inv_n
    v1 = st[3] * inv_n - m1 * m1
    wsm = jax.nn.softmax(fuse_w)
    a0 = wsm[0] * g0 * jax.lax.rsqrt(v0 + 1e-5)
    a1 = wsm[1] * g1 * jax.lax.rsqrt(v1 + 1e-5)
    par = jnp.concatenate(
        [a0[None], (wsm[0] * be0 - m0 * a0)[None],
         a1[None], (wsm[1] * be1 - m1 * a1)[None],
         jnp.zeros((4, HID), jnp.float32)], axis=0)
    bfp = jnp.broadcast_to(bf[None], (8, EMB))

    fn = pl.pallas_call(
        _fuse_body,
        grid=(grid,),
        in_specs=[
            pl.BlockSpec((BLK, HID), lambda i: (i, 0)),
            pl.BlockSpec((BLK, HID), lambda i: (i, 0)),
            pl.BlockSpec((8, HID), lambda i: (0, 0)),
            pl.BlockSpec((HID, EMB), lambda i: (0, 0)),
            pl.BlockSpec((8, EMB), lambda i: (0, 0)),
        ],
        out_specs=pl.BlockSpec((BLK, EMB), lambda i: (i, 0)),
        out_shape=jax.ShapeDtypeStruct((N_ITEMS, EMB), jnp.float32),
    )(z0, z1, par, Wf.astype(jnp.bfloat16), bfp)

    warm_gather, edge_agg = _sc_kernels()
    idx_pad = jnp.concatenate(
        [warm_idx, jnp.zeros((N_WARM_PAD - N_WARM,), jnp.int32)]
    ).reshape(N_WARM_PAD // 128, 128)
    warm = warm_gather(fn, idx_pad)

    sb = cnt[0, :NW + 1].astype(jnp.int32)
    s_al = (sb[:NW] // ALIGN) * ALIGN
    e_al = jnp.minimum(((sb[1:] + ALIGN - 1) // ALIGN) * ALIGN, NNZ_PAD)
    bounds = jnp.concatenate(
        [(s_al // 128), (e_al - s_al) // CHUNK]).reshape(4, 16)
    zer = jnp.zeros((U_PER_W, EMB), jnp.float32)
    rows_p = jnp.concatenate(
        [inter_rows, jnp.full((NNZ_PAD - NNZ,), jnp.int32(2 ** 30))])
    cols_p = jnp.concatenate(
        [inter_cols, jnp.zeros((NNZ_PAD - NNZ,), jnp.int32)])

    user_sum = edge_agg(rows_p, cols_p, bounds, zer, warm)

    user_norm = pl.pallas_call(
        _norm_body,
        grid=(N_USERS // BLK,),
        in_specs=[pl.BlockSpec((BLK, EMB), lambda i: (i, 0))],
        out_specs=pl.BlockSpec((BLK, EMB), lambda i: (i, 0)),
        out_shape=jax.ShapeDtypeStruct((N_USERS, EMB), jnp.float32),
    )(user_sum)

    return (user_norm, fn)
